# Initial kernel scaffold; baseline (speedup 1.0000x reference)
import jax
import jax.numpy as jnp
from jax.experimental import pallas as pl


def kernel(x, dictionary):
    raise NotImplementedError("write your pallas kernel here")


# speedup vs baseline: 1.3674x; 1.3674x over previous
import functools

import jax
import jax.numpy as jnp
from jax import lax
from jax.experimental import pallas as pl
from jax.experimental.pallas import tpu as pltpu

D_MODEL = 1024
N_CLUSTERS = 64
TOP_K = 8
BASE_TEMP = 2.0
SEQ_LEN = 2048
PRED_LEN = 512

_TEMP = BASE_TEMP * (1.0 + PRED_LEN / SEQ_LEN)
_TILE = 512


def _fused_kernel(x_ref, dt_ref, d_ref, common_ref, resid_ref, aux_ref,
                  q_scratch, *, n_rows, n_tiles):
    i = pl.program_id(0)
    x = x_ref[...]
    dt = dt_ref[...]
    scores = jnp.dot(x, dt, preferred_element_type=jnp.float32)
    st = scores * (1.0 / _TEMP)

    m = jnp.max(st, axis=-1, keepdims=True)
    e = jnp.exp(st - m)
    q = e / jnp.sum(e, axis=-1, keepdims=True)
    q_scratch[pl.ds(i * _TILE, _TILE), :] = q

    t, k = st.shape
    col = lax.broadcasted_iota(jnp.int32, (t, k), 1)
    neg_inf = jnp.float32(-jnp.inf)

    sw = st
    mask = jnp.zeros((t, k), dtype=jnp.bool_)
    for _ in range(TOP_K):
        mx = jnp.max(sw, axis=-1, keepdims=True)
        first = jnp.min(jnp.where(sw == mx, col, k), axis=-1, keepdims=True)
        sel = col == first
        sw = jnp.where(sel, neg_inf, sw)
        mask = jnp.logical_or(mask, sel)

    e2 = jnp.where(mask, e, 0.0)
    w = e2 / jnp.sum(e2, axis=-1, keepdims=True)

    x_common = jnp.dot(w, d_ref[...], preferred_element_type=jnp.float32)
    common_ref[...] = x_common
    resid_ref[...] = x - x_common

    @pl.when(i == n_tiles - 1)
    def _aux():
        qa = q_scratch[...]
        f = jnp.sum(qa, axis=0, keepdims=True)
        wgt = (qa * qa) / f
        p = wgt / jnp.sum(wgt, axis=-1, keepdims=True)
        kl = jnp.sum(p * (jnp.log(p) - jnp.log(qa))) / n_rows
        d = d_ref[...]
        gram = lax.dot_general(d, d, (((1,), (1,)), ((), ())),
                               preferred_element_type=jnp.float32)
        kk = gram.shape[0]
        eye = (lax.broadcasted_iota(jnp.int32, (kk, kk), 0)
               == lax.broadcasted_iota(jnp.int32, (kk, kk), 1))
        ortho = jnp.mean((gram - jnp.where(eye, 1.0, 0.0)) ** 2)
        aux = kl * (SEQ_LEN / PRED_LEN) + 0.1 * ortho
        aux_ref[...] = jnp.full((8, 128), aux, dtype=jnp.float32)


@jax.jit
def kernel(x, dictionary):
    b, n, d = x.shape
    k = dictionary.shape[0]
    rows = b * n
    n_tiles = rows // _TILE
    x2 = x.reshape(rows, d)
    dict_t = dictionary.T

    grid_spec = pltpu.PrefetchScalarGridSpec(
        num_scalar_prefetch=0,
        grid=(n_tiles,),
        in_specs=[
            pl.BlockSpec((_TILE, d), lambda i: (i, 0)),
            pl.BlockSpec((d, k), lambda i: (0, 0)),
            pl.BlockSpec((k, d), lambda i: (0, 0)),
        ],
        out_specs=[
            pl.BlockSpec((_TILE, d), lambda i: (i, 0)),
            pl.BlockSpec((_TILE, d), lambda i: (i, 0)),
            pl.BlockSpec((8, 128), lambda i: (0, 0)),
        ],
        scratch_shapes=[pltpu.VMEM((rows, k), jnp.float32)],
    )

    common, resid, aux = pl.pallas_call(
        functools.partial(_fused_kernel, n_rows=rows, n_tiles=n_tiles),
        grid_spec=grid_spec,
        out_shape=[
            jax.ShapeDtypeStruct((rows, d), jnp.float32),
            jax.ShapeDtypeStruct((rows, d), jnp.float32),
            jax.ShapeDtypeStruct((8, 128), jnp.float32),
        ],
    )(x2, dict_t, dictionary)

    return (common.reshape(b, n, d), resid.reshape(b, n, d),
            aux[0, 0])
